# Initial kernel scaffold; baseline (speedup 1.0000x reference)
import jax
import jax.numpy as jnp
from jax.experimental import pallas as pl


def kernel(x, edge_index, W1, b1, W2, b2, W3, b3):
    raise NotImplementedError("write your pallas kernel here")


# speedup vs baseline: 8.9637x; 8.9637x over previous
import functools

import jax
import jax.numpy as jnp
from jax import lax
from jax.experimental import pallas as pl
from jax.experimental.pallas import tpu as pltpu
from jax.experimental.pallas import tpu_sc as plsc

NC = 2
NS = 16
LANE = 128
IDXC = 16

N = 10000
N_PAD = 10112
ROW_BLK = 2000
D_IN = 128
H = 256


def _mesh():
    return plsc.VectorSubcoreMesh(core_axis_name="c", subcore_axis_name="s")


def _make_deg_kernel(nblk_total):
    nblk = nblk_total // (NC * NS)
    rows_per_tile = N_PAD // NS

    @functools.partial(
        pl.kernel,
        out_type=jax.ShapeDtypeStruct((NC, N_PAD, D_IN), jnp.float32),
        mesh=_mesh(),
        scratch_types=[
            pltpu.VMEM_SHARED((N_PAD, D_IN), jnp.float32),
            pltpu.VMEM((nblk, LANE), jnp.int32),
            pltpu.VMEM((LANE, D_IN), jnp.float32),
        ],
    )
    def deg_kernel(dst2d, zeros_hbm, ones_hbm, deg_out, acc, dstv, onesv):
        cid = lax.axis_index("c")
        sid = lax.axis_index("s")
        wid = cid * NS + sid
        pltpu.sync_copy(zeros_hbm, acc.at[pl.ds(sid * rows_per_tile, rows_per_tile)])
        pltpu.sync_copy(ones_hbm, onesv)
        pltpu.sync_copy(dst2d.at[pl.ds(wid * nblk, nblk)], dstv)
        plsc.subcore_barrier()

        def body(b, carry):
            pltpu.sync_copy(onesv, acc.at[dstv.at[b]], add=True)
            return carry

        lax.fori_loop(0, nblk, body, 0)
        plsc.subcore_barrier()
        pltpu.sync_copy(
            acc.at[pl.ds(sid * rows_per_tile, rows_per_tile)],
            deg_out.at[cid, pl.ds(sid * rows_per_tile, rows_per_tile)],
        )

    return deg_kernel


def _make_agg_edgesplit_kernel(fh, nblk_total):
    nblk = nblk_total // (NC * NS)
    nchunks = nblk // IDXC
    rows_per_tile = N_PAD // NS

    @functools.partial(
        pl.kernel,
        out_type=jax.ShapeDtypeStruct((NC, N_PAD, fh), jnp.float32),
        mesh=_mesh(),
        scratch_types=[
            pltpu.VMEM_SHARED((N_PAD, fh), jnp.float32),
            pltpu.VMEM((IDXC, LANE), jnp.int32),
            pltpu.VMEM((IDXC, LANE), jnp.int32),
            pltpu.VMEM((LANE, fh), jnp.float32),
            pltpu.VMEM((LANE, fh), jnp.float32),
            pltpu.SemaphoreType.DMA,
            pltpu.SemaphoreType.DMA,
        ],
    )
    def agge_kernel(hs_flat, src2d, dst2d, zeros_hbm, out, acc, srcv, dstv,
                    rows0, rows1, sem0, sem1):
        cid = lax.axis_index("c")
        sid = lax.axis_index("s")
        wid = cid * NS + sid
        pltpu.sync_copy(zeros_hbm, acc.at[pl.ds(sid * rows_per_tile, rows_per_tile)])
        plsc.subcore_barrier()

        bufs = (rows0, rows1)
        sems = (sem0, sem1)

        def chunk(ci, carry):
            base = wid * nblk + ci * IDXC
            pltpu.sync_copy(src2d.at[pl.ds(base, IDXC)], srcv)
            pltpu.sync_copy(dst2d.at[pl.ds(base, IDXC)], dstv)
            pltpu.async_copy(hs_flat.at[srcv.at[0]], rows0, sem0)
            pltpu.async_copy(hs_flat.at[srcv.at[1]], rows1, sem1)

            def body(i, c2):
                for b in range(2):
                    j = 2 * i + b
                    rows, sem = bufs[b], sems[b]
                    pltpu.make_async_copy(hs_flat.at[pl.ds(0, LANE)], rows, sem).wait()
                    pltpu.sync_copy(rows, acc.at[dstv.at[j]], add=True)

                    @pl.when(j + 2 < IDXC)
                    def _():
                        pltpu.async_copy(hs_flat.at[srcv.at[j + 2]], rows, sem)

                return c2

            lax.fori_loop(0, IDXC // 2, body, 0)
            return carry

        lax.fori_loop(0, nchunks, chunk, 0)
        plsc.subcore_barrier()
        pltpu.sync_copy(
            acc.at[pl.ds(sid * rows_per_tile, rows_per_tile)],
            out.at[cid, pl.ds(sid * rows_per_tile, rows_per_tile)],
        )

    return agge_kernel


def _make_agg_featsplit_kernel(fh, nblk_total):
    nblk = nblk_total // NS
    nchunks = nblk // IDXC
    init_rows = N_PAD // NS

    @functools.partial(
        pl.kernel,
        out_type=jax.ShapeDtypeStruct((NC, N_PAD, fh), jnp.float32),
        mesh=_mesh(),
        scratch_types=[
            pltpu.VMEM_SHARED((N_PAD, fh), jnp.float32),
            pltpu.VMEM((IDXC, LANE), jnp.int32),
            pltpu.VMEM((IDXC, LANE), jnp.int32),
            pltpu.VMEM((LANE, fh), jnp.float32),
            pltpu.VMEM((LANE, fh), jnp.float32),
            pltpu.SemaphoreType.DMA,
            pltpu.SemaphoreType.DMA,
        ],
    )
    def aggf_kernel(hs_flat, srcs3d, dst2d, out, acc, srcv, dstv, rows0, rows1,
                    sem0, sem1):
        cid = lax.axis_index("c")
        sid = lax.axis_index("s")
        pltpu.sync_copy(
            hs_flat.at[pl.ds(cid * N_PAD + sid * init_rows, init_rows)],
            acc.at[pl.ds(sid * init_rows, init_rows)],
        )
        plsc.subcore_barrier()

        bufs = (rows0, rows1)
        sems = (sem0, sem1)

        def chunk(ci, carry):
            base = sid * nblk + ci * IDXC
            pltpu.sync_copy(srcs3d.at[cid, pl.ds(base, IDXC)], srcv)
            pltpu.sync_copy(dst2d.at[pl.ds(base, IDXC)], dstv)
            pltpu.async_copy(hs_flat.at[srcv.at[0]], rows0, sem0)
            pltpu.async_copy(hs_flat.at[srcv.at[1]], rows1, sem1)

            def body(i, c2):
                for b in range(2):
                    j = 2 * i + b
                    rows, sem = bufs[b], sems[b]
                    pltpu.make_async_copy(hs_flat.at[pl.ds(0, LANE)], rows, sem).wait()
                    pltpu.sync_copy(rows, acc.at[dstv.at[j]], add=True)

                    @pl.when(j + 2 < IDXC)
                    def _():
                        pltpu.async_copy(hs_flat.at[srcv.at[j + 2]], rows, sem)

                return c2

            lax.fori_loop(0, IDXC // 2, body, 0)
            return carry

        lax.fori_loop(0, nchunks, chunk, 0)
        plsc.subcore_barrier()
        pltpu.sync_copy(
            acc.at[pl.ds(sid * init_rows, init_rows)],
            out.at[cid, pl.ds(sid * init_rows, init_rows)],
        )

    return aggf_kernel


def _tc1_body(x_ref, deg_ref, dis_ref, hs1_ref):
    d = deg_ref[0][:, 0:1] + deg_ref[1][:, 0:1] + 1.0
    dis = lax.rsqrt(d)
    dis_ref[...] = dis
    hs1_ref[...] = x_ref[...] * dis


def _tc2_body(g1_ref, hs1_ref, dis_ref, w1_ref, b1_ref, hs2_ref):
    dis = dis_ref[...]
    m = (g1_ref[0] + g1_ref[1] + hs1_ref[...]) * dis
    h = jnp.dot(m, w1_ref[...], preferred_element_type=jnp.float32) + b1_ref[...]
    hs = jnp.maximum(h, 0.0) * dis
    hs2_ref[0] = hs[:, : H // 2]
    hs2_ref[1] = hs[:, H // 2 :]


def _tc3_body(g2_ref, dis_ref, w2_ref, b2_ref, w3_ref, zs_ref):
    dis = dis_ref[...]
    m = jnp.concatenate([g2_ref[0], g2_ref[1]], axis=1) * dis
    h2 = jnp.maximum(
        jnp.dot(m, w2_ref[...], preferred_element_type=jnp.float32) + b2_ref[...],
        0.0,
    )
    z = jnp.dot(h2, w3_ref[...], preferred_element_type=jnp.float32)
    zs_ref[...] = jnp.broadcast_to(z * dis, (z.shape[0], D_IN))


def _tc4_body(g3_ref, zs_ref, dis_ref, b3_ref, out_ref):
    g = g3_ref[0][:, 0:1] + g3_ref[1][:, 0:1] + zs_ref[...][:, 0:1]
    out_ref[...] = jax.nn.sigmoid(dis_ref[...] * g + b3_ref[...])


def _row_spec(cols):
    return pl.BlockSpec((ROW_BLK, cols), lambda i: (i, 0))


def _split_spec(cols):
    return pl.BlockSpec((NC, ROW_BLK, cols), lambda i: (0, i, 0))


def _full_spec(r, c):
    return pl.BlockSpec((r, c), lambda i: (0, 0))


def kernel(x, edge_index, W1, b1, W2, b2, W3, b3):
    n = x.shape[0]
    assert n == N
    e = edge_index.shape[1]
    blk_edges = LANE * NC * NS * 8
    e_pad = -(-e // blk_edges) * blk_edges
    nblk_total = e_pad // LANE

    src = edge_index[0]
    dst = edge_index[1]
    pad = e_pad - e
    src_p = jnp.concatenate([src, jnp.zeros((pad,), jnp.int32)])
    dst_p = jnp.concatenate([dst, jnp.full((pad,), N, jnp.int32)])
    src2d = src_p.reshape(nblk_total, LANE)
    dst2d = dst_p.reshape(nblk_total, LANE)
    srcs3d = jnp.stack([src2d, src2d + N_PAD])

    zeros_wide = jnp.zeros((N_PAD // NS, D_IN), jnp.float32)
    ones_blk = jnp.ones((LANE, D_IN), jnp.float32)

    deg2 = _make_deg_kernel(nblk_total)(dst2d, zeros_wide, ones_blk)

    grid = (N // ROW_BLK,)
    dis, hs1 = pl.pallas_call(
        _tc1_body,
        grid=grid,
        in_specs=[_row_spec(D_IN), _split_spec(D_IN)],
        out_specs=[_row_spec(1), _row_spec(D_IN)],
        out_shape=[
            jax.ShapeDtypeStruct((N, 1), jnp.float32),
            jax.ShapeDtypeStruct((N_PAD, D_IN), jnp.float32),
        ],
    )(x, deg2)

    g1 = _make_agg_edgesplit_kernel(D_IN, nblk_total)(hs1, src2d, dst2d, zeros_wide)

    hs2 = pl.pallas_call(
        _tc2_body,
        grid=grid,
        in_specs=[
            _split_spec(D_IN),
            _row_spec(D_IN),
            _row_spec(1),
            _full_spec(D_IN, H),
            _full_spec(1, H),
        ],
        out_specs=_split_spec(H // 2),
        out_shape=jax.ShapeDtypeStruct((NC, N_PAD, H // 2), jnp.float32),
    )(g1, hs1, dis, W1, b1.reshape(1, H))

    g2 = _make_agg_featsplit_kernel(H // 2, nblk_total)(
        hs2.reshape(NC * N_PAD, H // 2), srcs3d, dst2d
    )

    zs = pl.pallas_call(
        _tc3_body,
        grid=grid,
        in_specs=[
            _split_spec(H // 2),
            _row_spec(1),
            _full_spec(H, H),
            _full_spec(1, H),
            _full_spec(H, 1),
        ],
        out_specs=_row_spec(D_IN),
        out_shape=jax.ShapeDtypeStruct((N_PAD, D_IN), jnp.float32),
    )(g2, dis, W2, b2.reshape(1, H), W3)

    g3 = _make_agg_edgesplit_kernel(D_IN, nblk_total)(zs, src2d, dst2d, zeros_wide)

    out = pl.pallas_call(
        _tc4_body,
        grid=grid,
        in_specs=[
            _split_spec(D_IN),
            _row_spec(D_IN),
            _row_spec(1),
            _full_spec(1, 1),
        ],
        out_specs=_row_spec(1),
        out_shape=jax.ShapeDtypeStruct((N, 1), jnp.float32),
    )(g3, zs, dis, b3.reshape(1, 1))

    return out


# speedup vs baseline: 10.8676x; 1.2124x over previous
import functools

import jax
import jax.numpy as jnp
from jax import lax
from jax.experimental import pallas as pl
from jax.experimental.pallas import tpu as pltpu
from jax.experimental.pallas import tpu_sc as plsc

NC = 2
NS = 16
LANE = 128
IDXC = 16

N = 10000
N_PAD = 10112
HR = 80
ROW_BLK = 2000
D_IN = 128
H = 256


def _mesh():
    return plsc.VectorSubcoreMesh(core_axis_name="c", subcore_axis_name="s")


def _make_hist_kernel(nblk_total, with_gather):
    nblk = nblk_total // (NC * NS)
    rpt = HR // NS

    scratch = [
        pltpu.VMEM_SHARED((NS, HR, LANE), jnp.float32),
        pltpu.VMEM((HR, LANE), jnp.float32),
        pltpu.VMEM((nblk, LANE), jnp.int32),
        pltpu.VMEM((rpt, LANE), jnp.float32),
        pltpu.VMEM((rpt, LANE), jnp.float32),
    ]
    if with_gather:
        scratch += [
            pltpu.VMEM((HR, LANE), jnp.float32),
            pltpu.VMEM((nblk, LANE), jnp.int32),
        ]

    @functools.partial(
        pl.kernel,
        out_type=jax.ShapeDtypeStruct((NC, NS, rpt, LANE), jnp.float32),
        mesh=_mesh(),
        compiler_params=pltpu.CompilerParams(needs_layout_passes=False),
        scratch_types=scratch,
    )
    def hist_kernel(*args):
        if with_gather:
            zflat, src2d, dst2d, out, shared, hist, dstv, fv, rv, zv, srcv = args
        else:
            dst2d, out, shared, hist, dstv, fv, rv = args
        cid = lax.axis_index("c")
        sid = lax.axis_index("s")
        wid = cid * NS + sid
        zero16 = jnp.zeros((16,), jnp.float32)
        ones16 = jnp.ones((16,), jnp.float32)

        def zf(r, c):
            for g in range(LANE // 16):
                hist[r, pl.ds(g * 16, 16)] = zero16
            return c

        lax.fori_loop(0, HR, zf, 0)
        if with_gather:
            pltpu.sync_copy(zflat, zv)
            pltpu.sync_copy(src2d.at[pl.ds(wid * nblk, nblk)], srcv)
        pltpu.sync_copy(dst2d.at[pl.ds(wid * nblk, nblk)], dstv)

        def body(b, c):
            for kk in range(8):
                d16 = dstv[b, pl.ds(kk * 16, 16)]
                if with_gather:
                    s16 = srcv[b, pl.ds(kk * 16, 16)]
                    v = plsc.load_gather(zv, [s16 >> 7, s16 & 127])
                else:
                    v = ones16
                plsc.addupdate_scatter(hist, [d16 >> 7, d16 & 127], v)
            return c

        lax.fori_loop(0, nblk, body, 0)

        pltpu.sync_copy(hist, shared.at[sid])
        plsc.subcore_barrier()

        for r in range(rpt):
            for g in range(LANE // 16):
                fv[r, pl.ds(g * 16, 16)] = zero16

        def addw(w, c):
            pltpu.sync_copy(shared.at[w, pl.ds(sid * rpt, rpt)], rv)
            for r in range(rpt):
                for g in range(LANE // 16):
                    fv[r, pl.ds(g * 16, 16)] += rv[r, pl.ds(g * 16, 16)]
            return c

        lax.fori_loop(0, NS, addw, 0)
        pltpu.sync_copy(fv, out.at[cid, sid])

    return hist_kernel


def _make_agg_edgesplit_kernel(fh, nblk_total):
    nblk = nblk_total // (NC * NS)
    nchunks = nblk // IDXC
    rows_per_tile = N_PAD // NS

    @functools.partial(
        pl.kernel,
        out_type=jax.ShapeDtypeStruct((NC, N_PAD, fh), jnp.float32),
        mesh=_mesh(),
        scratch_types=[
            pltpu.VMEM_SHARED((N_PAD, fh), jnp.float32),
            pltpu.VMEM((IDXC, LANE), jnp.int32),
            pltpu.VMEM((IDXC, LANE), jnp.int32),
            pltpu.VMEM((LANE, fh), jnp.float32),
            pltpu.VMEM((LANE, fh), jnp.float32),
            pltpu.SemaphoreType.DMA,
            pltpu.SemaphoreType.DMA,
        ],
    )
    def agge_kernel(hs_flat, src2d, dst2d, zeros_hbm, out, acc, srcv, dstv,
                    rows0, rows1, sem0, sem1):
        cid = lax.axis_index("c")
        sid = lax.axis_index("s")
        wid = cid * NS + sid
        pltpu.sync_copy(zeros_hbm, acc.at[pl.ds(sid * rows_per_tile, rows_per_tile)])
        plsc.subcore_barrier()

        bufs = (rows0, rows1)
        sems = (sem0, sem1)

        def chunk(ci, carry):
            base = wid * nblk + ci * IDXC
            pltpu.sync_copy(src2d.at[pl.ds(base, IDXC)], srcv)
            pltpu.sync_copy(dst2d.at[pl.ds(base, IDXC)], dstv)
            pltpu.async_copy(hs_flat.at[srcv.at[0]], rows0, sem0)
            pltpu.async_copy(hs_flat.at[srcv.at[1]], rows1, sem1)

            def body(i, c2):
                for b in range(2):
                    j = 2 * i + b
                    rows, sem = bufs[b], sems[b]
                    pltpu.make_async_copy(hs_flat.at[pl.ds(0, LANE)], rows, sem).wait()
                    pltpu.sync_copy(rows, acc.at[dstv.at[j]], add=True)

                    @pl.when(j + 2 < IDXC)
                    def _():
                        pltpu.async_copy(hs_flat.at[srcv.at[j + 2]], rows, sem)

                return c2

            lax.fori_loop(0, IDXC // 2, body, 0)
            return carry

        lax.fori_loop(0, nchunks, chunk, 0)
        plsc.subcore_barrier()
        pltpu.sync_copy(
            acc.at[pl.ds(sid * rows_per_tile, rows_per_tile)],
            out.at[cid, pl.ds(sid * rows_per_tile, rows_per_tile)],
        )

    return agge_kernel


def _make_agg_featsplit_kernel(fh, nblk_total):
    nblk = nblk_total // NS
    nchunks = nblk // IDXC
    init_rows = N_PAD // NS

    @functools.partial(
        pl.kernel,
        out_type=jax.ShapeDtypeStruct((NC, N_PAD, fh), jnp.float32),
        mesh=_mesh(),
        scratch_types=[
            pltpu.VMEM_SHARED((N_PAD, fh), jnp.float32),
            pltpu.VMEM((IDXC, LANE), jnp.int32),
            pltpu.VMEM((IDXC, LANE), jnp.int32),
            pltpu.VMEM((LANE, fh), jnp.float32),
            pltpu.VMEM((LANE, fh), jnp.float32),
            pltpu.SemaphoreType.DMA,
            pltpu.SemaphoreType.DMA,
        ],
    )
    def aggf_kernel(hs_flat, srcs3d, dst2d, out, acc, srcv, dstv, rows0, rows1,
                    sem0, sem1):
        cid = lax.axis_index("c")
        sid = lax.axis_index("s")
        pltpu.sync_copy(
            hs_flat.at[pl.ds(cid * N_PAD + sid * init_rows, init_rows)],
            acc.at[pl.ds(sid * init_rows, init_rows)],
        )
        plsc.subcore_barrier()

        bufs = (rows0, rows1)
        sems = (sem0, sem1)

        def chunk(ci, carry):
            base = sid * nblk + ci * IDXC
            pltpu.sync_copy(srcs3d.at[cid, pl.ds(base, IDXC)], srcv)
            pltpu.sync_copy(dst2d.at[pl.ds(base, IDXC)], dstv)
            pltpu.async_copy(hs_flat.at[srcv.at[0]], rows0, sem0)
            pltpu.async_copy(hs_flat.at[srcv.at[1]], rows1, sem1)

            def body(i, c2):
                for b in range(2):
                    j = 2 * i + b
                    rows, sem = bufs[b], sems[b]
                    pltpu.make_async_copy(hs_flat.at[pl.ds(0, LANE)], rows, sem).wait()
                    pltpu.sync_copy(rows, acc.at[dstv.at[j]], add=True)

                    @pl.when(j + 2 < IDXC)
                    def _():
                        pltpu.async_copy(hs_flat.at[srcv.at[j + 2]], rows, sem)

                return c2

            lax.fori_loop(0, IDXC // 2, body, 0)
            return carry

        lax.fori_loop(0, nchunks, chunk, 0)
        plsc.subcore_barrier()
        pltpu.sync_copy(
            acc.at[pl.ds(sid * init_rows, init_rows)],
            out.at[cid, pl.ds(sid * init_rows, init_rows)],
        )

    return aggf_kernel


def _tc1_body(x_ref, deg_ref, dis_ref, hs1_ref):
    d = deg_ref[0] + deg_ref[1] + 1.0
    dis = lax.rsqrt(d)
    dis_ref[...] = dis
    hs1_ref[...] = x_ref[...] * dis


def _tc2_body(g1_ref, hs1_ref, dis_ref, w1_ref, b1_ref, hs2_ref):
    dis = dis_ref[...]
    m = (g1_ref[0] + g1_ref[1] + hs1_ref[...]) * dis
    h = jnp.dot(m, w1_ref[...], preferred_element_type=jnp.float32) + b1_ref[...]
    hs = jnp.maximum(h, 0.0) * dis
    hs2_ref[0] = hs[:, : H // 2]
    hs2_ref[1] = hs[:, H // 2 :]


def _tc3_body(g2_ref, dis_ref, w2_ref, b2_ref, w3_ref, zs_ref):
    dis = dis_ref[...]
    m = jnp.concatenate([g2_ref[0], g2_ref[1]], axis=1) * dis
    h2 = jnp.maximum(
        jnp.dot(m, w2_ref[...], preferred_element_type=jnp.float32) + b2_ref[...],
        0.0,
    )
    z = jnp.dot(h2, w3_ref[...], preferred_element_type=jnp.float32)
    zs_ref[...] = z * dis


def _tc4_body(g3_ref, zs_ref, dis_ref, b3_ref, out_ref):
    g = g3_ref[0] + g3_ref[1] + zs_ref[...]
    out_ref[...] = jax.nn.sigmoid(dis_ref[...] * g + b3_ref[...])


def _row_spec(cols):
    return pl.BlockSpec((ROW_BLK, cols), lambda i: (i, 0))


def _split_spec(cols):
    return pl.BlockSpec((NC, ROW_BLK, cols), lambda i: (0, i, 0))


def _full_spec(r, c):
    return pl.BlockSpec((r, c), lambda i: (0, 0))


def kernel(x, edge_index, W1, b1, W2, b2, W3, b3):
    n = x.shape[0]
    assert n == N
    e = edge_index.shape[1]
    blk_edges = LANE * NC * NS * 8
    e_pad = -(-e // blk_edges) * blk_edges
    nblk_total = e_pad // LANE

    src = edge_index[0]
    dst = edge_index[1]
    pad = e_pad - e
    src_p = jnp.concatenate([src, jnp.zeros((pad,), jnp.int32)])
    dst_p = jnp.concatenate([dst, jnp.full((pad,), N, jnp.int32)])
    src2d = src_p.reshape(nblk_total, LANE)
    dst2d = dst_p.reshape(nblk_total, LANE)
    srcs3d = jnp.stack([src2d, src2d + N_PAD])

    zeros_wide = jnp.zeros((N_PAD // NS, D_IN), jnp.float32)

    deg_raw = _make_hist_kernel(nblk_total, with_gather=False)(dst2d)
    deg2 = deg_raw.reshape(NC, HR * LANE)[:, :N].reshape(NC, N, 1)

    grid = (N // ROW_BLK,)
    dis, hs1 = pl.pallas_call(
        _tc1_body,
        grid=grid,
        in_specs=[_row_spec(D_IN), _split_spec(1)],
        out_specs=[_row_spec(1), _row_spec(D_IN)],
        out_shape=[
            jax.ShapeDtypeStruct((N, 1), jnp.float32),
            jax.ShapeDtypeStruct((N_PAD, D_IN), jnp.float32),
        ],
    )(x, deg2)

    g1 = _make_agg_edgesplit_kernel(D_IN, nblk_total)(hs1, src2d, dst2d, zeros_wide)

    hs2 = pl.pallas_call(
        _tc2_body,
        grid=grid,
        in_specs=[
            _split_spec(D_IN),
            _row_spec(D_IN),
            _row_spec(1),
            _full_spec(D_IN, H),
            _full_spec(1, H),
        ],
        out_specs=_split_spec(H // 2),
        out_shape=jax.ShapeDtypeStruct((NC, N_PAD, H // 2), jnp.float32),
    )(g1, hs1, dis, W1, b1.reshape(1, H))

    g2 = _make_agg_featsplit_kernel(H // 2, nblk_total)(
        hs2.reshape(NC * N_PAD, H // 2), srcs3d, dst2d
    )

    zs = pl.pallas_call(
        _tc3_body,
        grid=grid,
        in_specs=[
            _split_spec(H // 2),
            _row_spec(1),
            _full_spec(H, H),
            _full_spec(1, H),
            _full_spec(H, 1),
        ],
        out_specs=_row_spec(1),
        out_shape=jax.ShapeDtypeStruct((N, 1), jnp.float32),
    )(g2, dis, W2, b2.reshape(1, H), W3)

    zflat = jnp.concatenate([zs[:, 0], jnp.zeros((HR * LANE - N,), jnp.float32)])
    g3_raw = _make_hist_kernel(nblk_total, with_gather=True)(
        zflat.reshape(HR, LANE), src2d, dst2d)
    g3 = g3_raw.reshape(NC, HR * LANE)[:, :N].reshape(NC, N, 1)

    out = pl.pallas_call(
        _tc4_body,
        grid=grid,
        in_specs=[
            _split_spec(1),
            _row_spec(1),
            _row_spec(1),
            _full_spec(1, 1),
        ],
        out_specs=_row_spec(1),
        out_shape=jax.ShapeDtypeStruct((N, 1), jnp.float32),
    )(g3, zs, dis, b3.reshape(1, 1))

    return out


# speedup vs baseline: 11.3445x; 1.0439x over previous
import functools

import jax
import jax.numpy as jnp
from jax import lax
from jax.experimental import pallas as pl
from jax.experimental.pallas import tpu as pltpu
from jax.experimental.pallas import tpu_sc as plsc

NC = 2
NS = 16
LANE = 128
IDXC = 16

N = 10000
N_PAD = 10112
HR = 80
ROW_BLK = 2000
D_IN = 128
H = 256


def _mesh():
    return plsc.VectorSubcoreMesh(core_axis_name="c", subcore_axis_name="s")


def _make_hist_kernel(nblk_total, with_gather):
    nblk = nblk_total // (NC * NS)
    rpt = HR // NS

    scratch = [
        pltpu.VMEM_SHARED((NS, HR, LANE), jnp.float32),
        pltpu.VMEM((HR, LANE), jnp.float32),
        pltpu.VMEM((nblk, LANE), jnp.int32),
        pltpu.VMEM((rpt, LANE), jnp.float32),
        pltpu.VMEM((rpt, LANE), jnp.float32),
    ]
    if with_gather:
        scratch += [
            pltpu.VMEM((HR, LANE), jnp.float32),
            pltpu.VMEM((nblk, LANE), jnp.int32),
        ]

    @functools.partial(
        pl.kernel,
        out_type=jax.ShapeDtypeStruct((NC, NS, rpt, LANE), jnp.float32),
        mesh=_mesh(),
        compiler_params=pltpu.CompilerParams(needs_layout_passes=False),
        scratch_types=scratch,
    )
    def hist_kernel(*args):
        if with_gather:
            zflat, src2d, dst2d, out, shared, hist, dstv, fv, rv, zv, srcv = args
        else:
            dst2d, out, shared, hist, dstv, fv, rv = args
        cid = lax.axis_index("c")
        sid = lax.axis_index("s")
        wid = cid * NS + sid
        zero16 = jnp.zeros((16,), jnp.float32)
        ones16 = jnp.ones((16,), jnp.float32)

        def zf(r, c):
            for g in range(LANE // 16):
                hist[r, pl.ds(g * 16, 16)] = zero16
            return c

        lax.fori_loop(0, HR, zf, 0)
        if with_gather:
            pltpu.sync_copy(zflat, zv)
            pltpu.sync_copy(src2d.at[pl.ds(wid * nblk, nblk)], srcv)
        pltpu.sync_copy(dst2d.at[pl.ds(wid * nblk, nblk)], dstv)

        def body(b, c):
            for kk in range(8):
                d16 = dstv[b, pl.ds(kk * 16, 16)]
                if with_gather:
                    s16 = srcv[b, pl.ds(kk * 16, 16)]
                    v = plsc.load_gather(zv, [s16 >> 7, s16 & 127])
                else:
                    v = ones16
                plsc.addupdate_scatter(hist, [d16 >> 7, d16 & 127], v)
            return c

        lax.fori_loop(0, nblk, body, 0)

        pltpu.sync_copy(hist, shared.at[sid])
        plsc.subcore_barrier()

        for r in range(rpt):
            for g in range(LANE // 16):
                fv[r, pl.ds(g * 16, 16)] = zero16

        def addw(w, c):
            pltpu.sync_copy(shared.at[w, pl.ds(sid * rpt, rpt)], rv)
            for r in range(rpt):
                for g in range(LANE // 16):
                    fv[r, pl.ds(g * 16, 16)] += rv[r, pl.ds(g * 16, 16)]
            return c

        lax.fori_loop(0, NS, addw, 0)
        pltpu.sync_copy(fv, out.at[cid, sid])

    return hist_kernel


def _make_agg_edgesplit_kernel(fh, nblk_total, nblk0=None):
    if nblk0 is None:
        nblk0 = nblk_total // 2
    nblk1 = nblk_total - nblk0
    nt0, nt1 = nblk0 // NS, nblk1 // NS
    rows_per_tile = N_PAD // NS

    @functools.partial(
        pl.kernel,
        out_type=jax.ShapeDtypeStruct((NC, N_PAD, fh), jnp.float32),
        mesh=_mesh(),
        scratch_types=[
            pltpu.VMEM_SHARED((N_PAD, fh), jnp.float32),
            pltpu.VMEM((IDXC, LANE), jnp.int32),
            pltpu.VMEM((IDXC, LANE), jnp.int32),
            pltpu.VMEM((LANE, fh), jnp.float32),
            pltpu.VMEM((LANE, fh), jnp.float32),
            pltpu.SemaphoreType.DMA,
            pltpu.SemaphoreType.DMA,
        ],
    )
    def agge_kernel(hs_flat, src2d, dst2d, zeros_hbm, out, acc, srcv, dstv,
                    rows0, rows1, sem0, sem1):
        cid = lax.axis_index("c")
        sid = lax.axis_index("s")
        wid = cid * NS + sid
        pltpu.sync_copy(zeros_hbm, acc.at[pl.ds(sid * rows_per_tile, rows_per_tile)])
        plsc.subcore_barrier()

        bufs = (rows0, rows1)
        sems = (sem0, sem1)

        def run_core(tile_base, nchunks):
            def chunk(ci, carry):
                base = tile_base + ci * IDXC
                pltpu.sync_copy(src2d.at[pl.ds(base, IDXC)], srcv)
                pltpu.sync_copy(dst2d.at[pl.ds(base, IDXC)], dstv)
                pltpu.async_copy(hs_flat.at[srcv.at[0]], rows0, sem0)
                pltpu.async_copy(hs_flat.at[srcv.at[1]], rows1, sem1)

                def body(i, c2):
                    for b in range(2):
                        j = 2 * i + b
                        rows, sem = bufs[b], sems[b]
                        pltpu.make_async_copy(hs_flat.at[pl.ds(0, LANE)], rows, sem).wait()
                        pltpu.sync_copy(rows, acc.at[dstv.at[j]], add=True)

                        @pl.when(j + 2 < IDXC)
                        def _():
                            pltpu.async_copy(hs_flat.at[srcv.at[j + 2]], rows, sem)

                    return c2

                lax.fori_loop(0, IDXC // 2, body, 0)
                return carry

            lax.fori_loop(0, nchunks, chunk, 0)

        @pl.when(cid == 0)
        def _():
            run_core(sid * nt0, nt0 // IDXC)

        @pl.when(cid == 1)
        def _():
            run_core(nblk0 + sid * nt1, nt1 // IDXC)

        plsc.subcore_barrier()
        pltpu.sync_copy(
            acc.at[pl.ds(sid * rows_per_tile, rows_per_tile)],
            out.at[cid, pl.ds(sid * rows_per_tile, rows_per_tile)],
        )

    return agge_kernel


def _make_agg_featsplit_kernel(fh, nblk_total):
    nblk = nblk_total // NS
    nchunks = nblk // IDXC
    init_rows = N_PAD // NS

    @functools.partial(
        pl.kernel,
        out_type=jax.ShapeDtypeStruct((NC, N_PAD, fh), jnp.float32),
        mesh=_mesh(),
        scratch_types=[
            pltpu.VMEM_SHARED((N_PAD, fh), jnp.float32),
            pltpu.VMEM((IDXC, LANE), jnp.int32),
            pltpu.VMEM((IDXC, LANE), jnp.int32),
            pltpu.VMEM((LANE, fh), jnp.float32),
            pltpu.VMEM((LANE, fh), jnp.float32),
            pltpu.SemaphoreType.DMA,
            pltpu.SemaphoreType.DMA,
        ],
    )
    def aggf_kernel(hs_flat, srcs3d, dst2d, out, acc, srcv, dstv, rows0, rows1,
                    sem0, sem1):
        cid = lax.axis_index("c")
        sid = lax.axis_index("s")
        pltpu.sync_copy(
            hs_flat.at[pl.ds(cid * N_PAD + sid * init_rows, init_rows)],
            acc.at[pl.ds(sid * init_rows, init_rows)],
        )
        plsc.subcore_barrier()

        bufs = (rows0, rows1)
        sems = (sem0, sem1)

        def chunk(ci, carry):
            base = sid * nblk + ci * IDXC
            pltpu.sync_copy(srcs3d.at[cid, pl.ds(base, IDXC)], srcv)
            pltpu.sync_copy(dst2d.at[pl.ds(base, IDXC)], dstv)
            pltpu.async_copy(hs_flat.at[srcv.at[0]], rows0, sem0)
            pltpu.async_copy(hs_flat.at[srcv.at[1]], rows1, sem1)

            def body(i, c2):
                for b in range(2):
                    j = 2 * i + b
                    rows, sem = bufs[b], sems[b]
                    pltpu.make_async_copy(hs_flat.at[pl.ds(0, LANE)], rows, sem).wait()
                    pltpu.sync_copy(rows, acc.at[dstv.at[j]], add=True)

                    @pl.when(j + 2 < IDXC)
                    def _():
                        pltpu.async_copy(hs_flat.at[srcv.at[j + 2]], rows, sem)

                return c2

            lax.fori_loop(0, IDXC // 2, body, 0)
            return carry

        lax.fori_loop(0, nchunks, chunk, 0)
        plsc.subcore_barrier()
        pltpu.sync_copy(
            acc.at[pl.ds(sid * init_rows, init_rows)],
            out.at[cid, pl.ds(sid * init_rows, init_rows)],
        )

    return aggf_kernel


def _tc1_body(x_ref, deg_ref, dis_ref, hs1_ref):
    d = deg_ref[0] + deg_ref[1] + 1.0
    dis = lax.rsqrt(d)
    dis_ref[...] = dis
    hs1_ref[...] = x_ref[...] * dis


def _tc2_body(g1_ref, hs1_ref, dis_ref, w1_ref, b1_ref, hs2_ref):
    dis = dis_ref[...]
    m = (g1_ref[0] + g1_ref[1] + hs1_ref[...]) * dis
    h = jnp.dot(m, w1_ref[...], preferred_element_type=jnp.float32) + b1_ref[...]
    hs = jnp.maximum(h, 0.0) * dis
    hs2_ref[0] = hs[:, : H // 2]
    hs2_ref[1] = hs[:, H // 2 :]


def _tc3_body(g2_ref, dis_ref, w2_ref, b2_ref, w3_ref, zs_ref):
    dis = dis_ref[...]
    m = jnp.concatenate([g2_ref[0], g2_ref[1]], axis=1) * dis
    h2 = jnp.maximum(
        jnp.dot(m, w2_ref[...], preferred_element_type=jnp.float32) + b2_ref[...],
        0.0,
    )
    z = jnp.dot(h2, w3_ref[...], preferred_element_type=jnp.float32)
    zs_ref[...] = z * dis


def _tc4_body(g3_ref, zs_ref, dis_ref, b3_ref, out_ref):
    g = g3_ref[0] + g3_ref[1] + zs_ref[...]
    out_ref[...] = jax.nn.sigmoid(dis_ref[...] * g + b3_ref[...])


def _row_spec(cols):
    return pl.BlockSpec((ROW_BLK, cols), lambda i: (i, 0))


def _split_spec(cols):
    return pl.BlockSpec((NC, ROW_BLK, cols), lambda i: (0, i, 0))


def _full_spec(r, c):
    return pl.BlockSpec((r, c), lambda i: (0, 0))


def kernel(x, edge_index, W1, b1, W2, b2, W3, b3):
    n = x.shape[0]
    assert n == N
    e = edge_index.shape[1]
    blk_edges = LANE * NC * NS * 8
    e_pad = -(-e // blk_edges) * blk_edges
    nblk_total = e_pad // LANE

    src = edge_index[0]
    dst = edge_index[1]
    pad = e_pad - e
    src_p = jnp.concatenate([src, jnp.zeros((pad,), jnp.int32)])
    dst_p = jnp.concatenate([dst, jnp.full((pad,), N, jnp.int32)])
    src2d = src_p.reshape(nblk_total, LANE)
    dst2d = dst_p.reshape(nblk_total, LANE)
    srcs3d = jnp.stack([src2d, src2d + N_PAD])

    zeros_wide = jnp.zeros((N_PAD // NS, D_IN), jnp.float32)

    deg_raw = _make_hist_kernel(nblk_total, with_gather=False)(dst2d)
    deg2 = deg_raw.reshape(NC, HR * LANE)[:, :N].reshape(NC, N, 1)

    grid = (N // ROW_BLK,)
    dis, hs1 = pl.pallas_call(
        _tc1_body,
        grid=grid,
        in_specs=[_row_spec(D_IN), _split_spec(1)],
        out_specs=[_row_spec(1), _row_spec(D_IN)],
        out_shape=[
            jax.ShapeDtypeStruct((N, 1), jnp.float32),
            jax.ShapeDtypeStruct((N_PAD, D_IN), jnp.float32),
        ],
    )(x, deg2)

    g1 = _make_agg_edgesplit_kernel(D_IN, nblk_total, nblk0=nblk_total * 4 // 5)(hs1, src2d, dst2d, zeros_wide)

    hs2 = pl.pallas_call(
        _tc2_body,
        grid=grid,
        in_specs=[
            _split_spec(D_IN),
            _row_spec(D_IN),
            _row_spec(1),
            _full_spec(D_IN, H),
            _full_spec(1, H),
        ],
        out_specs=_split_spec(H // 2),
        out_shape=jax.ShapeDtypeStruct((NC, N_PAD, H // 2), jnp.float32),
    )(g1, hs1, dis, W1, b1.reshape(1, H))

    g2 = _make_agg_featsplit_kernel(H // 2, nblk_total)(
        hs2.reshape(NC * N_PAD, H // 2), srcs3d, dst2d
    )

    zs = pl.pallas_call(
        _tc3_body,
        grid=grid,
        in_specs=[
            _split_spec(H // 2),
            _row_spec(1),
            _full_spec(H, H),
            _full_spec(1, H),
            _full_spec(H, 1),
        ],
        out_specs=_row_spec(1),
        out_shape=jax.ShapeDtypeStruct((N, 1), jnp.float32),
    )(g2, dis, W2, b2.reshape(1, H), W3)

    zflat = jnp.concatenate([zs[:, 0], jnp.zeros((HR * LANE - N,), jnp.float32)])
    g3_raw = _make_hist_kernel(nblk_total, with_gather=True)(
        zflat.reshape(HR, LANE), src2d, dst2d)
    g3 = g3_raw.reshape(NC, HR * LANE)[:, :N].reshape(NC, N, 1)

    out = pl.pallas_call(
        _tc4_body,
        grid=grid,
        in_specs=[
            _split_spec(1),
            _row_spec(1),
            _row_spec(1),
            _full_spec(1, 1),
        ],
        out_specs=_row_spec(1),
        out_shape=jax.ShapeDtypeStruct((N, 1), jnp.float32),
    )(g3, zs, dis, b3.reshape(1, 1))

    return out
